# baseline (speedup 1.0000x reference)
import functools

import jax
import jax.numpy as jnp
from jax import lax
from jax.experimental import pallas as pl
from jax.experimental.pallas import tpu as pltpu
from jax.experimental.pallas import tpu_sc as plsc

B, SEQ, D = 32, 196, 768
POOL, LENGTH, TOPK = 100, 10, 5
NCH = 4
CR = SEQ // NCH

mesh = plsc.VectorSubcoreMesh(core_axis_name="c", subcore_axis_name="s")


@functools.partial(
    pl.kernel,
    out_type=[
        jax.ShapeDtypeStruct((B, LENGTH + SEQ, D), jnp.float32),
        jax.ShapeDtypeStruct((1, 1), jnp.float32),
    ],
    mesh=mesh,
    scratch_types=[
        pltpu.VMEM((CR, D), jnp.float32),
        pltpu.VMEM((CR, D), jnp.float32),
        pltpu.SemaphoreType.DMA,
        pltpu.SemaphoreType.DMA,
    ],
    compiler_params=pltpu.CompilerParams(use_tc_tiling_on_sc=False),
)
def _sc_copy(x_hbm, xk_hbm, prompt_hbm, pk_hbm, out_hbm, rs_hbm,
             buf0, buf1, sem0, sem1):
    wid = lax.axis_index("s") * 2 + lax.axis_index("c")
    bufs = (buf0, buf1)
    sems = (sem0, sem1)
    cps = []
    for c in range(NCH):
        cp = pltpu.make_async_copy(
            x_hbm.at[wid, pl.ds(c * CR, CR), :], bufs[c % 2], sems[c % 2])
        if c >= 2:
            cps[c - 2][1].wait()
        cp.start()
        cps.append([cp, None])
        if c >= 1:
            cps[c - 1][0].wait()
            st = pltpu.make_async_copy(
                bufs[(c - 1) % 2],
                out_hbm.at[wid, pl.ds((c - 1) * CR + LENGTH, CR), :],
                sems[(c - 1) % 2])
            st.start()
            cps[c - 1][1] = st
    cps[NCH - 1][0].wait()
    st = pltpu.make_async_copy(
        bufs[(NCH - 1) % 2],
        out_hbm.at[wid, pl.ds((NCH - 1) * CR + LENGTH, CR), :],
        sems[(NCH - 1) % 2])
    st.start()
    cps[NCH - 1][1] = st
    cps[NCH - 2][1].wait()
    cps[NCH - 1][1].wait()


def kernel(x_embed, x_key, prompt, prompt_key):
    out, rs = _sc_copy(x_embed, x_key, prompt, prompt_key)
    return out, rs[0, 0]


# speedup vs baseline: 1.1365x; 1.1365x over previous
import functools

import jax
import jax.numpy as jnp
from jax import lax
from jax.experimental import pallas as pl
from jax.experimental.pallas import tpu as pltpu
from jax.experimental.pallas import tpu_sc as plsc

B, SEQ, D = 32, 196, 768
POOL, LENGTH, TOPK = 100, 10, 5
CW = 128
NCH = D // CW
NLC = CW // 16


def _mean_kernel(x_key_ref, prompt_ref, prompt_key_ref, mean_ref, rs_ref):
    xk = x_key_ref[...]
    xk = xk / jnp.maximum(
        jnp.sqrt(jnp.sum(xk * xk, axis=1, keepdims=True)), 1e-12)
    pk = prompt_key_ref[...]
    pk = pk / jnp.maximum(
        jnp.sqrt(jnp.sum(pk * pk, axis=1, keepdims=True)), 1e-12)

    sim = jnp.dot(xk, pk.T, preferred_element_type=jnp.float32)
    m = jnp.max(sim, axis=1, keepdims=True)
    e = jnp.exp(sim - m)
    p = e / jnp.sum(e, axis=1, keepdims=True)

    for l in range(LENGTH):
        mean_ref[:, l, :] = jnp.dot(
            p, prompt_ref[:, l, :],
            preferred_element_type=jnp.float32) * (1.0 / POOL)

    iota = jax.lax.broadcasted_iota(jnp.int32, (B, POOL), 1)
    v = sim
    total = jnp.float32(0.0)
    for _ in range(TOPK):
        mx = jnp.max(v, axis=1, keepdims=True)
        idx = jnp.min(jnp.where(v >= mx, iota, jnp.int32(POOL)),
                      axis=1, keepdims=True)
        total = total + jnp.sum(mx)
        v = jnp.where(iota == idx, -jnp.inf, v)
    rs_ref[...] = jnp.full((1, 1), total * (1.0 / B), jnp.float32)


_sc_mesh = plsc.VectorSubcoreMesh(core_axis_name="c", subcore_axis_name="s")


@functools.partial(
    pl.kernel,
    out_type=jax.ShapeDtypeStruct((B, LENGTH + SEQ, D), jnp.float32),
    mesh=_sc_mesh,
    scratch_types=[
        pltpu.VMEM((SEQ, CW), jnp.float32),
        pltpu.VMEM((LENGTH + SEQ, CW), jnp.float32),
        pltpu.VMEM((LENGTH, CW), jnp.float32),
    ],
)
def _sc_assemble(x_hbm, mean_hbm, out_hbm, xbuf, obuf, mbuf):
    wid = lax.axis_index("s") * 2 + lax.axis_index("c")
    for c in range(NCH):
        cols = pl.ds(c * CW, CW)
        pltpu.sync_copy(x_hbm.at[wid, :, cols], xbuf)
        pltpu.sync_copy(mean_hbm.at[wid, :, cols], mbuf)
        for l in range(LENGTH):
            for k in range(NLC):
                obuf[l, pl.ds(k * 16, 16)] = mbuf[l, pl.ds(k * 16, 16)]

        def _row(r, carry):
            for k in range(NLC):
                obuf[r + LENGTH, pl.ds(k * 16, 16)] = xbuf[r, pl.ds(k * 16, 16)]
            return carry

        lax.fori_loop(0, SEQ, _row, 0)
        pltpu.sync_copy(obuf, out_hbm.at[wid, :, cols])


@jax.jit
def kernel(x_embed, x_key, prompt, prompt_key):
    mean, rs = pl.pallas_call(
        _mean_kernel,
        in_specs=[
            pl.BlockSpec(memory_space=pltpu.MemorySpace.VMEM),
            pl.BlockSpec(memory_space=pltpu.MemorySpace.VMEM),
            pl.BlockSpec(memory_space=pltpu.MemorySpace.VMEM),
        ],
        out_specs=[
            pl.BlockSpec(memory_space=pltpu.MemorySpace.VMEM),
            pl.BlockSpec(memory_space=pltpu.MemorySpace.VMEM),
        ],
        out_shape=[
            jax.ShapeDtypeStruct((B, LENGTH, D), jnp.float32),
            jax.ShapeDtypeStruct((1, 1), jnp.float32),
        ],
    )(x_key, prompt, prompt_key)
    out = _sc_assemble(x_embed, mean)
    return out, rs[0, 0]
